# Initial kernel scaffold; baseline (speedup 1.0000x reference)
import jax
import jax.numpy as jnp
from jax.experimental import pallas as pl


def kernel(x, edge_index, W1, b1, W2, b2, Wout, bout):
    raise NotImplementedError("write your pallas kernel here")


# speedup vs baseline: 10.4978x; 10.4978x over previous
import functools

import jax
import jax.numpy as jnp
from jax import lax
from jax.experimental import pallas as pl
from jax.experimental.pallas import tpu as pltpu
from jax.experimental.pallas import tpu_sc as plsc

N = 10000
E = 320000
DEG_PAD = 10240
CHUNK = 128
NCHUNKS = E // CHUNK
INIT_CHUNK = 80
NICHUNKS = N // INIT_CHUNK
RB = 1000

_mesh = plsc.VectorSubcoreMesh(core_axis_name="c", subcore_axis_name="s")


@functools.partial(
    pl.kernel,
    out_type=jax.ShapeDtypeStruct((2 * DEG_PAD,), jnp.float32),
    mesh=_mesh,
    scratch_types=[
        pltpu.VMEM_SHARED((DEG_PAD,), jnp.float32),
        pltpu.VMEM((640,), jnp.float32),
        pltpu.VMEM((CHUNK,), jnp.int32),
        pltpu.VMEM((CHUNK,), jnp.float32),
    ],
)
def _deg_kernel(dst_hbm, out_hbm, acc, buf, idx, ones):
    c = lax.axis_index("c")
    s = lax.axis_index("s")
    wid = s * 2 + c
    for i in range(640 // 16):
        buf[pl.ds(i * 16, 16)] = jnp.zeros((16,), jnp.float32)
    for i in range(CHUNK // 16):
        ones[pl.ds(i * 16, 16)] = jnp.ones((16,), jnp.float32)
    pltpu.sync_copy(buf, acc.at[pl.ds(s * 640, 640)])
    plsc.subcore_barrier()

    def body(t, carry):
        cid = t * 32 + wid

        @pl.when(cid < NCHUNKS)
        def _():
            pltpu.sync_copy(dst_hbm.at[pl.ds(cid * CHUNK, CHUNK)], idx)
            pltpu.sync_copy(ones, acc.at[idx], add=True)

        return carry

    lax.fori_loop(0, (NCHUNKS + 31) // 32, body, 0)
    plsc.subcore_barrier()
    pltpu.sync_copy(acc.at[pl.ds(s * 640, 640)], buf)
    pltpu.sync_copy(buf, out_hbm.at[pl.ds(c * DEG_PAD + s * 640, 640)])


@functools.partial(
    pl.kernel,
    out_type=[jax.ShapeDtypeStruct((N, 128), jnp.float32),
              jax.ShapeDtypeStruct((N, 128), jnp.float32)],
    mesh=_mesh,
    scratch_types=[
        pltpu.VMEM_SHARED((N, 128), jnp.float32),
        pltpu.VMEM((INIT_CHUNK, 128), jnp.float32),
        pltpu.VMEM((CHUNK,), jnp.int32),
        pltpu.VMEM((CHUNK,), jnp.int32),
        pltpu.VMEM((CHUNK, 128), jnp.float32),
        pltpu.SemaphoreType.DMA,
    ],
)
def _agg_kernel(sa_hbm, sb_hbm, src_hbm, dst_hbm, outa_hbm, outb_hbm,
                acc, ibuf, isrc, idst, rows, sem):
    c = lax.axis_index("c")
    s = lax.axis_index("s")

    def work(s_hbm, out_hbm):
        def init_body(t, carry):
            cidx = t * 16 + s

            @pl.when(cidx < NICHUNKS)
            def _():
                r0 = cidx * INIT_CHUNK
                pltpu.sync_copy(s_hbm.at[pl.ds(r0, INIT_CHUNK)], ibuf)
                pltpu.sync_copy(ibuf, acc.at[pl.ds(r0, INIT_CHUNK)])

            return carry

        lax.fori_loop(0, (NICHUNKS + 15) // 16, init_body, 0)
        plsc.subcore_barrier()

        def body(t, carry):
            cid = t * 16 + s

            @pl.when(cid < NCHUNKS)
            def _():
                base = cid * CHUNK
                pltpu.sync_copy(src_hbm.at[pl.ds(base, CHUNK)], isrc)
                pltpu.sync_copy(dst_hbm.at[pl.ds(base, CHUNK)], idst)
                pltpu.async_copy(s_hbm.at[isrc], rows, sem).wait()
                pltpu.sync_copy(rows, acc.at[idst], add=True)

            return carry

        lax.fori_loop(0, (NCHUNKS + 15) // 16, body, 0)
        plsc.subcore_barrier()

        def drain_body(t, carry):
            cidx = t * 16 + s

            @pl.when(cidx < NICHUNKS)
            def _():
                r0 = cidx * INIT_CHUNK
                pltpu.sync_copy(acc.at[pl.ds(r0, INIT_CHUNK)], ibuf)
                pltpu.sync_copy(ibuf, out_hbm.at[pl.ds(r0, INIT_CHUNK)])

            return carry

        lax.fori_loop(0, (NICHUNKS + 15) // 16, drain_body, 0)

    @pl.when(c == 0)
    def _():
        work(sa_hbm, outa_hbm)

    @pl.when(c == 1)
    def _():
        work(sb_hbm, outb_hbm)


def _row(cols):
    return pl.BlockSpec((RB, cols), lambda i: (i, 0))


def _whole(shape):
    return pl.BlockSpec(shape, lambda i: (0,) * len(shape))


def _tc1_body(x_ref, rni_ref, dega_ref, degb_ref, w1_ref, sa_ref, sb_ref):
    dinv = lax.rsqrt(dega_ref[...] + degb_ref[...] + 1.0)
    hw = (jnp.dot(x_ref[...], w1_ref[:128, :], preferred_element_type=jnp.float32)
          + jnp.dot(rni_ref[...], w1_ref[128:, :], preferred_element_type=jnp.float32))
    sc = hw * dinv
    sa_ref[...] = sc[:, :128]
    sb_ref[...] = sc[:, 128:]


_tc1 = pl.pallas_call(
    _tc1_body,
    grid=(N // RB,),
    in_specs=[_row(128), _row(32), _row(1), _row(1), _whole((160, 256))],
    out_specs=[_row(128), _row(128)],
    out_shape=[jax.ShapeDtypeStruct((N, 128), jnp.float32)] * 2,
)


def _tc2_body(aa_ref, ab_ref, dega_ref, degb_ref, b_ref, w_ref, sa_ref, sb_ref):
    dinv = lax.rsqrt(dega_ref[...] + degb_ref[...] + 1.0)
    h_a = jnp.maximum(aa_ref[...] * dinv + b_ref[:, :128], 0.0)
    h_b = jnp.maximum(ab_ref[...] * dinv + b_ref[:, 128:], 0.0)
    s2 = (jnp.dot(h_a, w_ref[:128, :], preferred_element_type=jnp.float32)
          + jnp.dot(h_b, w_ref[128:, :], preferred_element_type=jnp.float32)) * dinv
    sa_ref[...] = s2[:, :128]
    sb_ref[...] = s2[:, 128:]


_tc2 = pl.pallas_call(
    _tc2_body,
    grid=(N // RB,),
    in_specs=[_row(128), _row(128), _row(1), _row(1),
              _whole((1, 256)), _whole((256, 256))],
    out_specs=[_row(128), _row(128)],
    out_shape=[jax.ShapeDtypeStruct((N, 128), jnp.float32)] * 2,
)


def _tc3_body(aa_ref, ab_ref, dega_ref, degb_ref, b_ref, w_ref, bout_ref, out_ref):
    dinv = lax.rsqrt(dega_ref[...] + degb_ref[...] + 1.0)
    h_a = jnp.maximum(aa_ref[...] * dinv + b_ref[:, :128], 0.0)
    h_b = jnp.maximum(ab_ref[...] * dinv + b_ref[:, 128:], 0.0)
    out_ref[...] = (jnp.dot(h_a, w_ref[:128, :], preferred_element_type=jnp.float32)
                    + jnp.dot(h_b, w_ref[128:, :], preferred_element_type=jnp.float32)
                    + bout_ref[...])


_tc3 = pl.pallas_call(
    _tc3_body,
    grid=(N // RB,),
    in_specs=[_row(128), _row(128), _row(1), _row(1),
              _whole((1, 256)), _whole((256, 256)), _whole((1, 256))],
    out_specs=_row(256),
    out_shape=jax.ShapeDtypeStruct((N, 256), jnp.float32),
)


def kernel(x, edge_index, W1, b1, W2, b2, Wout, bout):
    src = edge_index[0]
    dst = edge_index[1]
    rni = jax.random.normal(jax.random.key(42), (N, 32), dtype=jnp.float32)
    deg = _deg_kernel(dst)
    dega = deg[:N].reshape(N, 1)
    degb = deg[DEG_PAD:DEG_PAD + N].reshape(N, 1)
    b1r = b1.reshape(1, 256)
    b2r = b2.reshape(1, 256)
    boutr = bout.reshape(1, 256)
    s1a, s1b = _tc1(x, rni, dega, degb, W1)
    a1a, a1b = _agg_kernel(s1a, s1b, src, dst)
    s2a, s2b = _tc2(a1a, a1b, dega, degb, b1r, W2)
    a2a, a2b = _agg_kernel(s2a, s2b, src, dst)
    return _tc3(a2a, a2b, dega, degb, b2r, Wout, boutr)
